# Initial kernel scaffold; baseline (speedup 1.0000x reference)
import functools

import jax
import jax.numpy as jnp
from jax import lax
from jax.experimental import pallas as pl
from jax.experimental.pallas import tpu as pltpu
from jax.experimental.pallas import tpu_sc as plsc

NOISE = 0.01
L = 16
K = 128
NC = 2
NS = 16
NW = NC * NS


def _sc_segment(N, E, D):
    assert E % K == 0 and N % NS == 0 and D % L == 0
    nchunks = E // K
    rows_per_tile = N // NS
    zrows = 125 if rows_per_tile % 125 == 0 else rows_per_tile
    mesh = plsc.VectorSubcoreMesh(core_axis_name="c", subcore_axis_name="s")

    @functools.partial(
        pl.kernel,
        mesh=mesh,
        out_type=(
            jax.ShapeDtypeStruct((NC, N, D), jnp.float32),
            jax.ShapeDtypeStruct((NC, N, L), jnp.float32),
        ),
        scratch_types=[
            pltpu.VMEM((K,), jnp.int32),
            pltpu.VMEM((K,), jnp.int32),
            pltpu.VMEM((K,), jnp.float32),
            pltpu.VMEM((K, D), jnp.float32),
            pltpu.VMEM((K, L), jnp.float32),
            pltpu.VMEM_SHARED((N, D), jnp.float32),
            pltpu.VMEM_SHARED((N, L), jnp.float32),
            pltpu.SemaphoreType.DMA,
        ],
    )
    def k(allocs_hbm, ei_hbm, csi_hbm, out_rows, out_cnt,
          src_v, dst_v, csi_v, rows_v, cnt_v, acc_rows, acc_cnt, sem):
        c = lax.axis_index("c")
        s = lax.axis_index("s")
        wid = s * NC + c

        def zero_body(e, carry):
            for blk in range(D // L):
                rows_v[e, pl.ds(blk * L, L)] = jnp.zeros((L,), jnp.float32)
            cnt_v[e, :] = jnp.zeros((L,), jnp.float32)
            return carry
        lax.fori_loop(0, K, zero_body, 0)

        def zinit(j, carry):
            r0 = s * rows_per_tile + j * zrows
            pltpu.sync_copy(rows_v.at[pl.ds(0, zrows), :],
                            acc_rows.at[pl.ds(r0, zrows), :])
            pltpu.sync_copy(cnt_v.at[pl.ds(0, zrows), :],
                            acc_cnt.at[pl.ds(r0, zrows), :])
            return carry
        lax.fori_loop(0, rows_per_tile // zrows, zinit, 0)
        plsc.subcore_barrier()

        iota = lax.iota(jnp.int32, L)
        zidx = jnp.zeros((L,), jnp.int32)
        nch = nchunks // NW + jnp.where(wid < nchunks % NW, 1, 0)

        def chunk_body(i, carry):
            base = (wid + i * NW) * K
            pltpu.sync_copy(ei_hbm.at[0, pl.ds(base, K)], src_v)
            pltpu.sync_copy(ei_hbm.at[1, pl.ds(base, K)], dst_v)
            pltpu.sync_copy(csi_hbm.at[pl.ds(base, K)], csi_v)
            pltpu.async_copy(allocs_hbm.at[src_v], rows_v, sem).wait()

            def group_body(g, gcarry):
                sv = src_v[pl.ds(g * L, L)]
                dv = dst_v[pl.ds(g * L, L)]
                cv = csi_v[pl.ds(g * L, L)]
                m = jnp.where(sv != dv, 1.0, 0.0).astype(jnp.float32)
                w = cv * cv * m
                plsc.store_scatter(cnt_v, [g * L + iota, zidx], m)
                for lane in range(L):
                    e = g * L + lane
                    wl = w[lane]
                    for blk in range(D // L):
                        rows_v[e, pl.ds(blk * L, L)] = (
                            rows_v[e, pl.ds(blk * L, L)] * wl)
                return gcarry
            lax.fori_loop(0, K // L, group_body, 0)

            pltpu.sync_copy(rows_v, acc_rows.at[dst_v], add=True)
            pltpu.sync_copy(cnt_v, acc_cnt.at[dst_v], add=True)
            return carry
        lax.fori_loop(0, nch, chunk_body, 0)
        plsc.subcore_barrier()

        r0 = s * rows_per_tile
        pltpu.sync_copy(acc_rows.at[pl.ds(r0, rows_per_tile), :],
                        out_rows.at[c, pl.ds(r0, rows_per_tile), :])
        pltpu.sync_copy(acc_cnt.at[pl.ds(r0, rows_per_tile), :],
                        out_cnt.at[c, pl.ds(r0, rows_per_tile), :])

    return k


def _tc_finale(N, D, BN=1000):
    def body(rows_ref, cnt_ref, allocs_ref, scsi_ref, out_ref):
        i = pl.program_id(0)
        interf = rows_ref[0] + rows_ref[1] + NOISE
        cnt = cnt_ref[0, :, 0:1] + cnt_ref[1, :, 0:1]
        s2 = scsi_ref[...] * scsi_ref[...]
        rate = jnp.log2(1.0 + s2 * allocs_ref[...] / interf)
        rate = jnp.where(cnt > 0.0, rate, 0.0)

        @pl.when(i == 0)
        def _():
            out_ref[0, 0] = 0.0
        out_ref[0, 0] += jnp.sum(rate)

    return pl.pallas_call(
        body,
        grid=(N // BN,),
        in_specs=[
            pl.BlockSpec((NC, BN, D), lambda i: (0, i, 0)),
            pl.BlockSpec((NC, BN, L), lambda i: (0, i, 0)),
            pl.BlockSpec((BN, D), lambda i: (i, 0)),
            pl.BlockSpec((BN, 1), lambda i: (i, 0)),
        ],
        out_specs=pl.BlockSpec((1, 1), lambda i: (0, 0)),
        out_shape=jax.ShapeDtypeStruct((1, 1), jnp.float32),
    )


@jax.jit
def kernel(allocs, node_csi, edge_csi, edge_index):
    N, D = allocs.shape
    E = edge_csi.shape[0]
    rows_p, cnt_p = _sc_segment(N, E, D)(allocs, edge_index, edge_csi)
    tot = _tc_finale(N, D)(rows_p, cnt_p, allocs, node_csi)
    return -tot[0, 0] / (N * D)


# speedup vs baseline: 5.7020x; 5.7020x over previous
import functools

import jax
import jax.numpy as jnp
from jax import lax
from jax.experimental import pallas as pl
from jax.experimental.pallas import tpu as pltpu
from jax.experimental.pallas import tpu_sc as plsc

NOISE = 0.01
L = 16
K = 128
NC = 2
NS = 16
NW = NC * NS


def _sc_segment(N, E, D):
    assert E % K == 0 and D % L == 0
    nchunks = E // K
    NP = ((N + NS * K - 1) // (NS * K)) * (NS * K)
    rows_per_tile = NP // NS
    mesh = plsc.VectorSubcoreMesh(core_axis_name="c", subcore_axis_name="s")

    @functools.partial(
        pl.kernel,
        mesh=mesh,
        out_type=(
            jax.ShapeDtypeStruct((NC, NP, D), jnp.float32),
            jax.ShapeDtypeStruct((NC, NP, L), jnp.float32),
        ),
        scratch_types=[
            pltpu.VMEM((K,), jnp.int32),
            pltpu.VMEM((K,), jnp.int32),
            pltpu.VMEM((K,), jnp.float32),
            pltpu.VMEM((K,), jnp.int32),
            pltpu.VMEM((K, D), jnp.float32),
            pltpu.VMEM((K, L), jnp.float32),
            pltpu.VMEM_SHARED((NP, D), jnp.float32),
            pltpu.VMEM_SHARED((NP, L), jnp.float32),
            pltpu.SemaphoreType.DMA,
        ],
    )
    def k(allocs_hbm, srcs_hbm, dsts_hbm, csi_hbm, out_rows, out_cnt,
          src_v, dst_v, csi_v, idx_v, rows_v, cnt_v, acc_rows, acc_cnt, sem):
        c = lax.axis_index("c")
        s = lax.axis_index("s")
        wid = s * NC + c
        iota = lax.iota(jnp.int32, L)
        onehot0 = jnp.where(iota == 0, 1.0, 0.0).astype(jnp.float32)

        def zero_body(e, carry):
            for blk in range(D // L):
                rows_v[e, pl.ds(blk * L, L)] = jnp.zeros((L,), jnp.float32)
            cnt_v[e, :] = jnp.zeros((L,), jnp.float32)
            return carry
        lax.fori_loop(0, K, zero_body, 0)

        def fill_idx(r0):
            for gg in range(K // L):
                idx_v[pl.ds(gg * L, L)] = r0 + gg * L + iota

        def zinit(j, carry):
            fill_idx(s * rows_per_tile + j * K)
            pltpu.sync_copy(rows_v, acc_rows.at[idx_v])
            pltpu.sync_copy(cnt_v, acc_cnt.at[idx_v])
            return carry
        lax.fori_loop(0, rows_per_tile // K, zinit, 0)
        plsc.subcore_barrier()

        nch = nchunks // NW + jnp.where(wid < nchunks % NW, 1, 0)

        def chunk_body(i, carry):
            base = (wid + i * NW) * K
            pltpu.sync_copy(srcs_hbm.at[pl.ds(base, K)], src_v)
            pltpu.sync_copy(dsts_hbm.at[pl.ds(base, K)], dst_v)
            pltpu.sync_copy(csi_hbm.at[pl.ds(base, K)], csi_v)
            pltpu.async_copy(allocs_hbm.at[src_v], rows_v, sem).wait()

            def group_body(g, gcarry):
                sv = src_v[pl.ds(g * L, L)]
                dv = dst_v[pl.ds(g * L, L)]
                cv = csi_v[pl.ds(g * L, L)]
                m = jnp.where(sv != dv, 1.0, 0.0).astype(jnp.float32)
                w = cv * cv * m
                for lane in range(L):
                    e = g * L + lane
                    wl = w[lane]
                    cnt_v[e, :] = onehot0 * m[lane]
                    for blk in range(D // L):
                        rows_v[e, pl.ds(blk * L, L)] = (
                            rows_v[e, pl.ds(blk * L, L)] * wl)
                return gcarry
            lax.fori_loop(0, K // L, group_body, 0)

            pltpu.sync_copy(rows_v, acc_rows.at[dst_v], add=True)
            pltpu.sync_copy(cnt_v, acc_cnt.at[dst_v], add=True)
            return carry
        lax.fori_loop(0, nch, chunk_body, 0)
        plsc.subcore_barrier()

        def rback(j, carry):
            r0 = s * rows_per_tile + j * K
            fill_idx(r0)
            pltpu.sync_copy(acc_rows.at[idx_v], rows_v)
            pltpu.sync_copy(acc_cnt.at[idx_v], cnt_v)
            pltpu.sync_copy(rows_v, out_rows.at[c, pl.ds(r0, K), :])
            pltpu.sync_copy(cnt_v, out_cnt.at[c, pl.ds(r0, K), :])
            return carry
        lax.fori_loop(0, rows_per_tile // K, rback, 0)

    return k


def _tc_finale(N, D, BN=1000):
    def body(rows_ref, cnt_ref, allocs_ref, scsi_ref, out_ref):
        i = pl.program_id(0)
        interf = rows_ref[0] + rows_ref[1] + NOISE
        cnt = cnt_ref[0, :, 0:1] + cnt_ref[1, :, 0:1]
        s2 = scsi_ref[...] * scsi_ref[...]
        rate = jnp.log2(1.0 + s2 * allocs_ref[...] / interf)
        rate = jnp.where(cnt > 0.0, rate, 0.0)

        @pl.when(i == 0)
        def _():
            out_ref[...] = jnp.zeros((1, 1), jnp.float32)
        out_ref[...] += jnp.sum(rate).reshape(1, 1)

    return pl.pallas_call(
        body,
        grid=(N // BN,),
        in_specs=[
            pl.BlockSpec((NC, BN, D), lambda i: (0, i, 0)),
            pl.BlockSpec((NC, BN, L), lambda i: (0, i, 0)),
            pl.BlockSpec((BN, D), lambda i: (i, 0)),
            pl.BlockSpec((BN, 1), lambda i: (i, 0)),
        ],
        out_specs=pl.BlockSpec((1, 1), lambda i: (0, 0)),
        out_shape=jax.ShapeDtypeStruct((1, 1), jnp.float32),
    )


@jax.jit
def kernel(allocs, node_csi, edge_csi, edge_index):
    N, D = allocs.shape
    E = edge_csi.shape[0]
    rows_p, cnt_p = _sc_segment(N, E, D)(
        allocs, edge_index[0], edge_index[1], edge_csi)
    tot = _tc_finale(N, D)(rows_p, cnt_p, allocs, node_csi)
    return -tot[0, 0] / (N * D)


# speedup vs baseline: 9.9172x; 1.7393x over previous
import functools

import jax
import jax.numpy as jnp
from jax import lax
from jax.experimental import pallas as pl
from jax.experimental.pallas import tpu as pltpu
from jax.experimental.pallas import tpu_sc as plsc

NOISE = 0.01
L = 16
K = 64
NC = 2
NS = 16
NW = NC * NS


def _sc_segment(N, E, D):
    assert E % NW == 0 and D % L == 0 and K % L == 0
    EPW = E // NW
    NCH = EPW // K
    TAIL = EPW - NCH * K
    assert TAIL % L == 0 and EPW % 8 == 0
    NP = ((N + NS * K - 1) // (NS * K)) * (NS * K)
    rows_per_tile = NP // NS
    mesh = plsc.VectorSubcoreMesh(core_axis_name="c", subcore_axis_name="s")

    @functools.partial(
        pl.kernel,
        mesh=mesh,
        out_type=(
            jax.ShapeDtypeStruct((NC, NP, D), jnp.float32),
            jax.ShapeDtypeStruct((NC, NP, L), jnp.float32),
        ),
        scratch_types=[
            [pltpu.VMEM((K,), jnp.int32) for _ in range(2)],
            [pltpu.VMEM((K,), jnp.int32) for _ in range(2)],
            [pltpu.VMEM((K,), jnp.float32) for _ in range(2)],
            [pltpu.VMEM((K, D), jnp.float32) for _ in range(2)],
            [pltpu.VMEM((K, L), jnp.float32) for _ in range(2)],
            [pltpu.VMEM((K,), jnp.int32) for _ in range(2)],
            pltpu.VMEM((K,), jnp.int32),
            pltpu.VMEM((max(TAIL, L),), jnp.int32),
            [pltpu.SemaphoreType.DMA for _ in range(2)],
            [pltpu.SemaphoreType.DMA for _ in range(2)],
            [pltpu.SemaphoreType.DMA for _ in range(2)],
            pltpu.VMEM_SHARED((NP, D), jnp.float32),
            pltpu.VMEM_SHARED((NP, L), jnp.float32),
        ],
    )
    def k(allocs_hbm, srcs_hbm, dsts_hbm, csi_hbm, out_rows, out_cnt,
          srcb, dstb, csib, rows_b, cnt_b, sdst, idx_v, dstt,
          sl, sg, ss, acc_rows, acc_cnt):
        c = lax.axis_index("c")
        s = lax.axis_index("s")
        wid = s * NC + c
        base_w = wid * EPW
        iota = lax.iota(jnp.int32, L)
        onehot0 = jnp.where(iota == 0, 1.0, 0.0).astype(jnp.float32)

        def zero_body(e, carry):
            for blk in range(D // L):
                rows_b[0][e, pl.ds(blk * L, L)] = jnp.zeros((L,), jnp.float32)
            cnt_b[0][e, :] = jnp.zeros((L,), jnp.float32)
            return carry
        lax.fori_loop(0, K, zero_body, 0)

        def fill_idx(r0):
            for gg in range(K // L):
                idx_v[pl.ds(gg * L, L)] = r0 + gg * L + iota

        def zinit(j, carry):
            fill_idx(s * rows_per_tile + j * K)
            pltpu.sync_copy(rows_b[0], acc_rows.at[idx_v])
            pltpu.sync_copy(cnt_b[0], acc_cnt.at[idx_v])
            return carry
        lax.fori_loop(0, rows_per_tile // K, zinit, 0)

        def issue_loads(i, q):
            b = base_w + i * K
            pltpu.make_async_copy(
                srcs_hbm.at[pl.ds(b, K)], srcb[q], sl[q]).start()
            pltpu.make_async_copy(
                dsts_hbm.at[pl.ds(b, K)], dstb[q], sl[q]).start()
            pltpu.make_async_copy(
                csi_hbm.at[pl.ds(b, K)], csib[q], sl[q]).start()

        def wait_loads(q):
            pltpu.make_async_copy(
                srcs_hbm.at[pl.ds(0, K)], srcb[q], sl[q]).wait()
            pltpu.make_async_copy(
                dsts_hbm.at[pl.ds(0, K)], dstb[q], sl[q]).wait()
            pltpu.make_async_copy(
                csi_hbm.at[pl.ds(0, K)], csib[q], sl[q]).wait()

        def issue_gather(p):
            pltpu.make_async_copy(
                allocs_hbm.at[srcb[p]], rows_b[p], sg[p]).start()

        def wait_gather(p):
            pltpu.make_async_copy(
                allocs_hbm.at[srcb[p]], rows_b[p], sg[p]).wait()

        def issue_scatter(p):
            pltpu.sync_copy(rows_b[p], acc_rows.at[sdst[p]], add=True)
            pltpu.sync_copy(cnt_b[p], acc_cnt.at[sdst[p]], add=True)

        def wait_scatter(p):
            pass

        def compute(p):
            def group_body(g, gcarry):
                e0 = g * L
                sv = srcb[p][pl.ds(e0, L)]
                dv = dstb[p][pl.ds(e0, L)]
                cv = csib[p][pl.ds(e0, L)]
                m = jnp.where(sv != dv, 1.0, 0.0).astype(jnp.float32)
                w = cv * cv * m
                sdst[p][pl.ds(e0, L)] = dv
                for lane in range(L):
                    e = g * L + lane
                    wl = w[lane]
                    cnt_b[p][e, :] = onehot0 * m[lane]
                    for blk in range(D // L):
                        rows_b[p][e, pl.ds(blk * L, L)] = (
                            rows_b[p][e, pl.ds(blk * L, L)] * wl)
                return gcarry
            lax.fori_loop(0, K // L, group_body, 0)

        issue_loads(0, 0)
        issue_loads(1, 1)
        wait_loads(0)
        issue_gather(0)
        plsc.subcore_barrier()

        def chunk_step(i, p):
            wait_gather(p)

            @pl.when(i + 1 < NCH)
            def _():
                wait_loads(1 - p)
                issue_gather(1 - p)
            compute(p)

            @pl.when(i + 2 < NCH)
            def _():
                issue_loads(i + 2, p)
            issue_scatter(p)

        def pair_body(it, carry):
            chunk_step(it * 2, 0)
            chunk_step(it * 2 + 1, 1)
            return carry
        lax.fori_loop(0, NCH // 2, pair_body, 0)
        for i in range(NCH // 2 * 2, NCH):
            chunk_step(jnp.int32(i), i % 2)

        if TAIL:
            tb = base_w + NCH * K
            pltpu.sync_copy(srcs_hbm.at[pl.ds(tb, TAIL)],
                            srcb[0].at[pl.ds(0, TAIL)])
            pltpu.sync_copy(dsts_hbm.at[pl.ds(tb, TAIL)],
                            dstb[0].at[pl.ds(0, TAIL)])
            pltpu.sync_copy(csi_hbm.at[pl.ds(tb, TAIL)],
                            csib[0].at[pl.ds(0, TAIL)])
            pltpu.make_async_copy(
                allocs_hbm.at[srcb[0].at[pl.ds(0, TAIL)]],
                rows_b[0].at[pl.ds(0, TAIL), :], sg[0]).start()
            pltpu.make_async_copy(
                allocs_hbm.at[srcb[0].at[pl.ds(0, TAIL)]],
                rows_b[0].at[pl.ds(0, TAIL), :], sg[0]).wait()

            def tail_group(g, gcarry):
                e0 = g * L
                sv = srcb[0][pl.ds(e0, L)]
                dv = dstb[0][pl.ds(e0, L)]
                cv = csib[0][pl.ds(e0, L)]
                m = jnp.where(sv != dv, 1.0, 0.0).astype(jnp.float32)
                w = cv * cv * m
                dstt[pl.ds(e0, L)] = dv
                for lane in range(L):
                    e = g * L + lane
                    wl = w[lane]
                    cnt_b[0][e, :] = onehot0 * m[lane]
                    for blk in range(D // L):
                        rows_b[0][e, pl.ds(blk * L, L)] = (
                            rows_b[0][e, pl.ds(blk * L, L)] * wl)
                return gcarry
            lax.fori_loop(0, TAIL // L, tail_group, 0)
            pltpu.sync_copy(rows_b[0].at[pl.ds(0, TAIL), :],
                            acc_rows.at[dstt], add=True)
            pltpu.sync_copy(cnt_b[0].at[pl.ds(0, TAIL), :],
                            acc_cnt.at[dstt], add=True)

        plsc.subcore_barrier()

        def rback(j, carry):
            r0 = s * rows_per_tile + j * K
            fill_idx(r0)
            pltpu.sync_copy(acc_rows.at[idx_v], rows_b[0])
            pltpu.sync_copy(acc_cnt.at[idx_v], cnt_b[0])
            pltpu.sync_copy(rows_b[0], out_rows.at[c, pl.ds(r0, K), :])
            pltpu.sync_copy(cnt_b[0], out_cnt.at[c, pl.ds(r0, K), :])
            return carry
        lax.fori_loop(0, rows_per_tile // K, rback, 0)

    return k


def _tc_finale(N, D, BN=1000):
    def body(rows_ref, cnt_ref, allocs_ref, scsi_ref, out_ref):
        i = pl.program_id(0)
        interf = rows_ref[0] + rows_ref[1] + NOISE
        cnt = cnt_ref[0, :, 0:1] + cnt_ref[1, :, 0:1]
        s2 = scsi_ref[...] * scsi_ref[...]
        rate = jnp.log2(1.0 + s2 * allocs_ref[...] / interf)
        rate = jnp.where(cnt > 0.0, rate, 0.0)

        @pl.when(i == 0)
        def _():
            out_ref[...] = jnp.zeros((1, 1), jnp.float32)
        out_ref[...] += jnp.sum(rate).reshape(1, 1)

    return pl.pallas_call(
        body,
        grid=(N // BN,),
        in_specs=[
            pl.BlockSpec((NC, BN, D), lambda i: (0, i, 0)),
            pl.BlockSpec((NC, BN, L), lambda i: (0, i, 0)),
            pl.BlockSpec((BN, D), lambda i: (i, 0)),
            pl.BlockSpec((BN, 1), lambda i: (i, 0)),
        ],
        out_specs=pl.BlockSpec((1, 1), lambda i: (0, 0)),
        out_shape=jax.ShapeDtypeStruct((1, 1), jnp.float32),
    )


@jax.jit
def kernel(allocs, node_csi, edge_csi, edge_index):
    N, D = allocs.shape
    E = edge_csi.shape[0]
    rows_p, cnt_p = _sc_segment(N, E, D)(
        allocs, edge_index[0], edge_index[1], edge_csi)
    tot = _tc_finale(N, D)(rows_p, cnt_p, allocs, node_csi)
    return -tot[0, 0] / (N * D)
